# baseline (speedup 1.0000x reference)
import jax
import jax.numpy as jnp
from jax import lax
from jax.experimental import pallas as pl
from jax.experimental.pallas import tpu as pltpu
from jax.experimental.pallas import tpu_sc as plsc

HID = 1024
INTER = 512
NE = 8
T = 4096
P = 2 * T
BG = 256
G = 40
QP = G * BG
RB = 512
NW = 32


def _router_kernel(x_ref, gw_ref, logits_ref, ep_ref, wp_ref):
    x = x_ref[...]
    logits = jnp.dot(x, gw_ref[...], preferred_element_type=jnp.float32)
    logits_ref[...] = logits
    m = jnp.max(logits, axis=-1, keepdims=True)
    p = jnp.exp(logits - m)
    rw = p / jnp.sum(p, axis=-1, keepdims=True)
    ids = lax.broadcasted_iota(jnp.int32, rw.shape, 1)
    i1 = jnp.argmax(rw, axis=-1, keepdims=True)
    v1 = jnp.max(rw, axis=-1, keepdims=True)
    masked = jnp.where(ids == i1, -1.0, rw)
    i2 = jnp.argmax(masked, axis=-1, keepdims=True)
    v2 = jnp.max(masked, axis=-1, keepdims=True)
    denom = v1 + v2
    ep_ref[...] = jnp.concatenate([i1, i2], axis=1)
    wp_ref[...] = jnp.concatenate([v1 / denom, v2 / denom], axis=1)


def _run_router(x, gate_w):
    return pl.pallas_call(
        _router_kernel,
        grid=(T // RB,),
        in_specs=[
            pl.BlockSpec((RB, HID), lambda t: (t, 0)),
            pl.BlockSpec((HID, NE), lambda t: (0, 0)),
        ],
        out_specs=[
            pl.BlockSpec((RB, NE), lambda t: (t, 0)),
            pl.BlockSpec((RB, 2), lambda t: (t, 0)),
            pl.BlockSpec((RB, 2), lambda t: (t, 0)),
        ],
        out_shape=[
            jax.ShapeDtypeStruct((T, NE), jnp.float32),
            jax.ShapeDtypeStruct((T, 2), jnp.int32),
            jax.ShapeDtypeStruct((T, 2), jnp.float32),
        ],
    )(x, gate_w)


_NPB = 2 * T // RB


def _pair_cols(ep_ref, wp_ref, j):
    lane2 = lax.broadcasted_iota(jnp.int32, (RB, 2), 1)
    sel = (lane2 == j).astype(jnp.float32)
    e_col = jnp.sum(ep_ref[...].astype(jnp.float32) * sel, axis=1, keepdims=True)
    w_col = jnp.sum(wp_ref[...] * sel, axis=1, keepdims=True)
    lane8 = lax.broadcasted_iota(jnp.int32, (RB, NE), 1).astype(jnp.float32)
    oh = (e_col == lane8).astype(jnp.float32)
    return e_col, w_col, oh


def _scan_kernel(
    ep_ref, wp_ref, tri_ref, dest_ref, wj_ref, meta_ref, cnt_ref, cpre_ref, offs_ref
):
    b = pl.program_id(0)

    @pl.when(b == 0)
    def _init():
        cnt_ref[...] = jnp.zeros((1, NE), jnp.float32)

    @pl.when(b < _NPB)
    def _pass1():
        j = b // (T // RB)
        _, _, oh = _pair_cols(ep_ref, wp_ref, j)
        cpre_ref[pl.ds(b, 1), :] = cnt_ref[...]
        cnt_ref[...] += jnp.sum(oh, axis=0, keepdims=True)

    @pl.when(b == _NPB - 1)
    def _finish():
        counts = cnt_ref[...]
        padded = jnp.floor((counts + (BG - 1)) / BG) * BG
        acc = jnp.zeros((1, 1), jnp.float32)
        cols = []
        for e in range(NE):
            acc = acc + padded[:, e : e + 1]
            cols.append(acc)
        ends = jnp.concatenate(cols, axis=1)
        offs_ref[...] = ends - padded
        lane128 = lax.broadcasted_iota(jnp.int32, (1, 128), 1)
        bpos = lane128.astype(jnp.float32) * BG
        be = jnp.zeros((1, 128), jnp.float32)
        for e in range(NE):
            be += (bpos >= ends[:, e : e + 1]).astype(jnp.float32)
        be = jnp.clip(be, 0.0, NE - 1.0)
        used = jnp.floor((ends[:, NE - 1 : NE] + (BG - 1)) / BG)
        nblk = jnp.broadcast_to(used, (1, 128))
        meta_ref[...] = jnp.concatenate([be, nblk], axis=0).astype(jnp.int32)

    @pl.when(b >= _NPB)
    def _pass2():
        b2 = b - _NPB
        j = b2 // (T // RB)
        _, w_col, oh = _pair_cols(ep_ref, wp_ref, j)
        excl = jnp.dot(tri_ref[...], oh, preferred_element_type=jnp.float32)
        rank_local = jnp.sum(oh * excl, axis=1, keepdims=True)
        carried = jnp.sum(oh * cpre_ref[pl.ds(b2, 1), :], axis=1, keepdims=True)
        obase = jnp.sum(oh * offs_ref[...], axis=1, keepdims=True)
        dest_ref[...] = (rank_local + carried + obase).astype(jnp.int32)
        wj_ref[...] = w_col


def _run_scan(e_pair, w_pair):
    tri = (
        lax.broadcasted_iota(jnp.int32, (RB, RB), 0)
        > lax.broadcasted_iota(jnp.int32, (RB, RB), 1)
    ).astype(jnp.float32)
    return pl.pallas_call(
        _scan_kernel,
        grid=(2 * _NPB,),
        in_specs=[
            pl.BlockSpec((RB, 2), lambda b: (b % (T // RB), 0)),
            pl.BlockSpec((RB, 2), lambda b: (b % (T // RB), 0)),
            pl.BlockSpec((RB, RB), lambda b: (0, 0)),
        ],
        out_specs=[
            pl.BlockSpec((RB, 1), lambda b: (jnp.maximum(b - _NPB, 0), 0)),
            pl.BlockSpec((RB, 1), lambda b: (jnp.maximum(b - _NPB, 0), 0)),
            pl.BlockSpec((2, 128), lambda b: (0, 0)),
        ],
        out_shape=[
            jax.ShapeDtypeStruct((P, 1), jnp.int32),
            jax.ShapeDtypeStruct((P, 1), jnp.float32),
            jax.ShapeDtypeStruct((2, 128), jnp.int32),
        ],
        scratch_shapes=[
            pltpu.VMEM((1, NE), jnp.float32),
            pltpu.VMEM((_NPB, NE), jnp.float32),
            pltpu.VMEM((1, NE), jnp.float32),
        ],
        compiler_params=pltpu.CompilerParams(
            dimension_semantics=("arbitrary",),
        ),
    )(e_pair, w_pair, tri)


def _sc_mesh():
    return plsc.VectorSubcoreMesh(core_axis_name="c", subcore_axis_name="s")


_PPW = P // 16
_SPW = QP // 16


def _sc_scatter_body(
    dest_hbm, wj_hbm,
    tok_hbm, ws_hbm,
    dest_v, w_v, tok_v, zi_v, zf_v, tok_s, ws_s,
):
    cid = lax.axis_index("c")
    sid = lax.axis_index("s")

    @pl.when(cid == 0)
    def _zero():
        for c in range(_SPW // 16):
            zi_v[pl.ds(c * 16, 16)] = jnp.zeros((16,), jnp.int32)
            zf_v[pl.ds(c * 16, 16)] = jnp.zeros((16,), jnp.float32)
        slot = sid * _SPW
        pltpu.sync_copy(zi_v, tok_s.at[pl.ds(slot, _SPW)])
        pltpu.sync_copy(zf_v, ws_s.at[pl.ds(slot, _SPW)])

    plsc.subcore_barrier()

    @pl.when(cid == 0)
    def _scatter():
        base = sid * _PPW
        pltpu.sync_copy(dest_hbm.at[pl.ds(base, _PPW)], dest_v)
        pltpu.sync_copy(wj_hbm.at[pl.ds(base, _PPW)], w_v)
        for c in range(_PPW // 16):
            pvec = base + c * 16 + lax.iota(jnp.int32, 16)
            tok_v[pl.ds(c * 16, 16)] = jnp.bitwise_and(pvec, T - 1)
        pltpu.sync_copy(tok_v, tok_s.at[dest_v], add=True)
        pltpu.sync_copy(w_v, ws_s.at[dest_v], add=True)

    plsc.subcore_barrier()

    @pl.when(cid == 0)
    def _emit():
        slot = sid * _SPW
        pltpu.sync_copy(tok_s.at[pl.ds(slot, _SPW)], tok_hbm.at[pl.ds(slot, _SPW)])
        pltpu.sync_copy(ws_s.at[pl.ds(slot, _SPW)], ws_hbm.at[pl.ds(slot, _SPW)])


def _run_scatter(dest, w_j):
    f = pl.kernel(
        _sc_scatter_body,
        out_type=(
            jax.ShapeDtypeStruct((QP,), jnp.int32),
            jax.ShapeDtypeStruct((QP,), jnp.float32),
        ),
        mesh=_sc_mesh(),
        scratch_types=[
            pltpu.VMEM((_PPW,), jnp.int32),
            pltpu.VMEM((_PPW,), jnp.float32),
            pltpu.VMEM((_PPW,), jnp.int32),
            pltpu.VMEM((_SPW,), jnp.int32),
            pltpu.VMEM((_SPW,), jnp.float32),
            pltpu.VMEM_SHARED((QP,), jnp.int32),
            pltpu.VMEM_SHARED((QP,), jnp.float32),
        ],
    )
    return f(dest, w_j)


_RPW = QP // NW
_GCH = 16


def _sc_gather_body(
    x_hbm, tok_hbm, xs_hbm, idx_v, r0, r1, r2, r3, sem_g, sem_w
):
    cid = lax.axis_index("c")
    sid = lax.axis_index("s")
    wid = sid * 2 + cid
    rbase = wid * _RPW
    nch = _RPW // _GCH
    bufs = (r0, r1, r2, r3)
    pltpu.sync_copy(tok_hbm.at[pl.ds(rbase, _RPW)], idx_v)

    def _start(k):
        return pltpu.async_copy(
            x_hbm.at[idx_v.at[pl.ds(k * _GCH, _GCH)]], bufs[k % 4], sem_g
        )

    gathers = [None] * nch
    writes = [None] * nch
    for k in range(3):
        gathers[k] = _start(k)
    for k in range(nch):
        gathers[k].wait()
        if k >= 1:
            writes[k - 1].wait()
        if k + 3 < nch:
            gathers[k + 3] = _start(k + 3)
        writes[k] = pltpu.async_copy(
            bufs[k % 4], xs_hbm.at[pl.ds(rbase + k * _GCH, _GCH)], sem_w
        )
    writes[nch - 1].wait()


def _run_gather(x, tok):
    f = pl.kernel(
        _sc_gather_body,
        out_type=jax.ShapeDtypeStruct((QP, HID), jnp.float32),
        mesh=_sc_mesh(),
        scratch_types=[
            pltpu.VMEM((_RPW,), jnp.int32),
            pltpu.VMEM((_GCH, HID), jnp.float32),
            pltpu.VMEM((_GCH, HID), jnp.float32),
            pltpu.VMEM((_GCH, HID), jnp.float32),
            pltpu.VMEM((_GCH, HID), jnp.float32),
            pltpu.SemaphoreType.DMA,
            pltpu.SemaphoreType.DMA,
        ],
    )
    return f(x, tok)


def _gmm_kernel(m_ref, xs_ref, ws_ref, gp_ref, up_ref, dp_ref, ys_ref):
    b = pl.program_id(0)
    nblk = m_ref[1, 0]

    @pl.when(b < nblk)
    def _compute():
        x = xs_ref[...]
        g = jnp.dot(x, gp_ref[0], preferred_element_type=jnp.float32)
        u = jnp.dot(x, up_ref[0], preferred_element_type=jnp.float32)
        h = (g * jax.nn.sigmoid(g)) * u * ws_ref[...]
        ys_ref[...] = jnp.dot(h, dp_ref[0], preferred_element_type=jnp.float32)


def _run_gmm(meta, xs, ws_col, gate_proj_w, up_proj_w, down_proj_w):
    grid_spec = pltpu.PrefetchScalarGridSpec(
        num_scalar_prefetch=1,
        grid=(G,),
        in_specs=[
            pl.BlockSpec((BG, HID), lambda b, m: (b, 0)),
            pl.BlockSpec((BG, 1), lambda b, m: (b, 0)),
            pl.BlockSpec((1, HID, INTER), lambda b, m: (m[0, b], 0, 0)),
            pl.BlockSpec((1, HID, INTER), lambda b, m: (m[0, b], 0, 0)),
            pl.BlockSpec((1, INTER, HID), lambda b, m: (m[0, b], 0, 0)),
        ],
        out_specs=pl.BlockSpec((BG, HID), lambda b, m: (b, 0)),
    )
    return pl.pallas_call(
        _gmm_kernel,
        grid_spec=grid_spec,
        out_shape=jax.ShapeDtypeStruct((QP, HID), jnp.float32),
        compiler_params=pltpu.CompilerParams(
            dimension_semantics=("arbitrary",),
        ),
    )(meta, xs, ws_col, gate_proj_w, up_proj_w, down_proj_w)


_TPW = T // NW
_MCH = 16


def _sc_mix_body(
    ys_hbm, dest_hbm, out_hbm, i0_v, i1_v, a0_v, b0_v, a1_v, b1_v, sem_g, sem_w
):
    cid = lax.axis_index("c")
    sid = lax.axis_index("s")
    wid = sid * 2 + cid
    tbase = wid * _TPW
    nch = _TPW // _MCH
    abufs = (a0_v, a1_v)
    bbufs = (b0_v, b1_v)
    pltpu.sync_copy(dest_hbm.at[pl.ds(tbase, _TPW)], i0_v)
    pltpu.sync_copy(dest_hbm.at[pl.ds(T + tbase, _TPW)], i1_v)

    def _start(k):
        ds = pl.ds(k * _MCH, _MCH)
        ga = pltpu.async_copy(ys_hbm.at[i0_v.at[ds]], abufs[k % 2], sem_g)
        gb = pltpu.async_copy(ys_hbm.at[i1_v.at[ds]], bbufs[k % 2], sem_g)
        return ga, gb

    gathers = [None] * nch
    writes = [None] * nch
    gathers[0] = _start(0)
    for k in range(nch):
        a_v = abufs[k % 2]
        b_v = bbufs[k % 2]
        gathers[k][0].wait()
        gathers[k][1].wait()
        if k >= 1:
            writes[k - 1].wait()
        if k + 1 < nch:
            gathers[k + 1] = _start(k + 1)

        def _add(i, carry):
            r = i // 16
            g = i - r * 16
            for u in range(4):
                ds = pl.ds((g * 4 + u) * 16, 16)
                a_v[r, ds] = a_v[r, ds] + b_v[r, ds]
            return carry

        lax.fori_loop(0, _MCH * 16, _add, 0)
        writes[k] = pltpu.async_copy(
            a_v, out_hbm.at[pl.ds(tbase + k * _MCH, _MCH)], sem_w
        )
    writes[nch - 1].wait()


def _run_mix(ys, dest):
    f = pl.kernel(
        _sc_mix_body,
        out_type=jax.ShapeDtypeStruct((T, HID), jnp.float32),
        mesh=_sc_mesh(),
        scratch_types=[
            pltpu.VMEM((_TPW,), jnp.int32),
            pltpu.VMEM((_TPW,), jnp.int32),
            pltpu.VMEM((_MCH, HID), jnp.float32),
            pltpu.VMEM((_MCH, HID), jnp.float32),
            pltpu.VMEM((_MCH, HID), jnp.float32),
            pltpu.VMEM((_MCH, HID), jnp.float32),
            pltpu.SemaphoreType.DMA,
            pltpu.SemaphoreType.DMA,
        ],
    )
    return f(ys, dest)


def kernel(hidden_states, gate_w, gate_proj_w, up_proj_w, down_proj_w):
    batch, seq_len, dim = hidden_states.shape
    x = hidden_states.reshape(-1, dim)

    logits, e_pair, w_pair = _run_router(x, gate_w)
    dest, w_j, meta = _run_scan(e_pair, w_pair)
    tok, ws = _run_scatter(dest.reshape(P), w_j.reshape(P))
    xs = _run_gather(x, tok)
    ys = _run_gmm(meta, xs, ws.reshape(QP, 1), gate_proj_w, up_proj_w, down_proj_w)
    out = _run_mix(ys, dest.reshape(P))
    return out.reshape(batch, seq_len, dim), logits


# speedup vs baseline: 1.8827x; 1.8827x over previous
import jax
import jax.numpy as jnp
from jax.experimental import pallas as pl
from jax.experimental.pallas import tpu as pltpu

HID = 1024
INTER = 512
NE = 8
TB = 256


def _router_kernel(x_ref, gw_ref, logits_ref, rw_ref):
    x = x_ref[...]
    logits = jnp.dot(x, gw_ref[...], preferred_element_type=jnp.float32)
    logits_ref[...] = logits
    m = jnp.max(logits, axis=-1, keepdims=True)
    p = jnp.exp(logits - m)
    rw = p / jnp.sum(p, axis=-1, keepdims=True)
    ids = jax.lax.broadcasted_iota(jnp.int32, rw.shape, 1)
    i1 = jnp.argmax(rw, axis=-1, keepdims=True)
    v1 = jnp.max(rw, axis=-1, keepdims=True)
    masked = jnp.where(ids == i1, -1.0, rw)
    i2 = jnp.argmax(masked, axis=-1, keepdims=True)
    v2 = jnp.max(masked, axis=-1, keepdims=True)
    denom = v1 + v2
    rw_ref[...] = jnp.where(ids == i1, v1 / denom, 0.0) + jnp.where(
        ids == i2, v2 / denom, 0.0
    )


def _moe_kernel(x_ref, rw_ref, gp_ref, up_ref, dp_ref, out_ref, h_ref):
    x = x_ref[...].astype(jnp.bfloat16)
    rw = rw_ref[...]
    for e in range(NE):
        g = jnp.dot(x, gp_ref[e], preferred_element_type=jnp.float32)
        u = jnp.dot(x, up_ref[e], preferred_element_type=jnp.float32)
        w = rw[:, e : e + 1]
        h_ref[:, e * INTER : (e + 1) * INTER] = (
            (g * jax.nn.sigmoid(g)) * u * w
        ).astype(jnp.bfloat16)
    out_ref[...] = jnp.dot(h_ref[...], dp_ref[...], preferred_element_type=jnp.float32)


def kernel(hidden_states, gate_w, gate_proj_w, up_proj_w, down_proj_w):
    batch, seq_len, dim = hidden_states.shape
    x = hidden_states.reshape(-1, dim)
    T = x.shape[0]

    logits, rw = pl.pallas_call(
        _router_kernel,
        grid=(T // TB,),
        in_specs=[
            pl.BlockSpec((TB, HID), lambda t: (t, 0)),
            pl.BlockSpec((HID, NE), lambda t: (0, 0)),
        ],
        out_specs=[
            pl.BlockSpec((TB, NE), lambda t: (t, 0)),
            pl.BlockSpec((TB, NE), lambda t: (t, 0)),
        ],
        out_shape=[
            jax.ShapeDtypeStruct((T, NE), jnp.float32),
            jax.ShapeDtypeStruct((T, NE), jnp.float32),
        ],
    )(x, gate_w)

    out = pl.pallas_call(
        _moe_kernel,
        grid=(T // TB,),
        in_specs=[
            pl.BlockSpec((TB, HID), lambda t: (t, 0)),
            pl.BlockSpec((TB, NE), lambda t: (t, 0)),
            pl.BlockSpec((NE, HID, INTER), lambda t: (0, 0, 0)),
            pl.BlockSpec((NE, HID, INTER), lambda t: (0, 0, 0)),
            pl.BlockSpec((NE * INTER, HID), lambda t: (0, 0)),
        ],
        out_specs=pl.BlockSpec((TB, HID), lambda t: (t, 0)),
        out_shape=jax.ShapeDtypeStruct((T, HID), jnp.float32),
        scratch_shapes=[pltpu.VMEM((TB, NE * INTER), jnp.bfloat16)],
        compiler_params=pltpu.CompilerParams(
            dimension_semantics=("arbitrary",),
            vmem_limit_bytes=100 * 1024 * 1024,
        ),
    )(
        x,
        rw,
        gate_proj_w.astype(jnp.bfloat16),
        up_proj_w.astype(jnp.bfloat16),
        down_proj_w.reshape(NE * INTER, HID).astype(jnp.bfloat16),
    )

    return out.reshape(batch, seq_len, dim), logits


# speedup vs baseline: 2.1017x; 1.1163x over previous
import jax
import jax.numpy as jnp
from jax.experimental import pallas as pl
from jax.experimental.pallas import tpu as pltpu

HID = 1024
INTER = 512
NE = 8
TB = 256


def _router_kernel(x_ref, gw_ref, logits_ref, rw_ref):
    x = x_ref[...]
    logits = jnp.dot(x, gw_ref[...], preferred_element_type=jnp.float32)
    logits_ref[...] = logits
    m = jnp.max(logits, axis=-1, keepdims=True)
    p = jnp.exp(logits - m)
    rw = p / jnp.sum(p, axis=-1, keepdims=True)
    ids = jax.lax.broadcasted_iota(jnp.int32, rw.shape, 1)
    i1 = jnp.argmax(rw, axis=-1, keepdims=True)
    v1 = jnp.max(rw, axis=-1, keepdims=True)
    masked = jnp.where(ids == i1, -1.0, rw)
    i2 = jnp.argmax(masked, axis=-1, keepdims=True)
    v2 = jnp.max(masked, axis=-1, keepdims=True)
    denom = v1 + v2
    rw_ref[...] = jnp.where(ids == i1, v1 / denom, 0.0) + jnp.where(
        ids == i2, v2 / denom, 0.0
    )


def _moe_kernel(x_ref, rw_ref, gp_ref, up_ref, dp_ref, out_ref, h_ref):
    x = x_ref[...]
    rw = rw_ref[...]
    for e in range(NE):
        g = jnp.dot(x, gp_ref[e], preferred_element_type=jnp.float32)
        u = jnp.dot(x, up_ref[e], preferred_element_type=jnp.float32)
        w = rw[:, e : e + 1]
        h_ref[:, e * INTER : (e + 1) * INTER] = (g * jax.nn.sigmoid(g)) * u * w
    out_ref[...] = jnp.dot(h_ref[...], dp_ref[...], preferred_element_type=jnp.float32)


def kernel(hidden_states, gate_w, gate_proj_w, up_proj_w, down_proj_w):
    batch, seq_len, dim = hidden_states.shape
    x = hidden_states.reshape(-1, dim)
    T = x.shape[0]

    logits, rw = pl.pallas_call(
        _router_kernel,
        grid=(T // TB,),
        in_specs=[
            pl.BlockSpec((TB, HID), lambda t: (t, 0)),
            pl.BlockSpec((HID, NE), lambda t: (0, 0)),
        ],
        out_specs=[
            pl.BlockSpec((TB, NE), lambda t: (t, 0)),
            pl.BlockSpec((TB, NE), lambda t: (t, 0)),
        ],
        out_shape=[
            jax.ShapeDtypeStruct((T, NE), jnp.float32),
            jax.ShapeDtypeStruct((T, NE), jnp.float32),
        ],
    )(x, gate_w)

    out = pl.pallas_call(
        _moe_kernel,
        grid=(T // TB,),
        in_specs=[
            pl.BlockSpec((TB, HID), lambda t: (t, 0)),
            pl.BlockSpec((TB, NE), lambda t: (t, 0)),
            pl.BlockSpec((NE, HID, INTER), lambda t: (0, 0, 0)),
            pl.BlockSpec((NE, HID, INTER), lambda t: (0, 0, 0)),
            pl.BlockSpec((NE * INTER, HID), lambda t: (0, 0)),
        ],
        out_specs=pl.BlockSpec((TB, HID), lambda t: (t, 0)),
        out_shape=jax.ShapeDtypeStruct((T, HID), jnp.float32),
        scratch_shapes=[pltpu.VMEM((TB, NE * INTER), jnp.float32)],
        compiler_params=pltpu.CompilerParams(
            dimension_semantics=("arbitrary",),
            vmem_limit_bytes=100 * 1024 * 1024,
        ),
    )(x, rw, gate_proj_w, up_proj_w, down_proj_w.reshape(NE * INTER, HID))

    return out.reshape(batch, seq_len, dim), logits
